# Initial kernel scaffold; baseline (speedup 1.0000x reference)
import jax
import jax.numpy as jnp
from jax.experimental import pallas as pl


def kernel(x, edge_index, edge_attrs, gW1, gb1, gW2, gb2, lW0, lb0, lW1, lb1, lW2, lb2, lW3, lb3):
    raise NotImplementedError("write your pallas kernel here")


# speedup vs baseline: 10.4986x; 10.4986x over previous
import functools
import jax
import jax.numpy as jnp
from jax import lax
from jax.experimental import pallas as pl
from jax.experimental.pallas import tpu as pltpu
from jax.experimental.pallas import tpu_sc as plsc

N = 10000
E = 320000
D_IN = 128
H = 64
D_OUT = 128

NC = 2
NS = 16
NW = NC * NS
L = 16

EPW = E // NW
K = 80
NCHUNK = EPW // K
RPT = N // NS

_mesh = plsc.VectorSubcoreMesh(
    core_axis_name="c", subcore_axis_name="s", num_cores=NC, num_subcores=NS)


@functools.partial(
    pl.kernel,
    mesh=_mesh,
    out_type=jax.ShapeDtypeStruct((NW, N), jnp.float32),
    scratch_types=[
        pltpu.VMEM((N,), jnp.float32),
        pltpu.VMEM((K,), jnp.int32),
        pltpu.VMEM((K,), jnp.float32),
    ],
    compiler_params=pltpu.CompilerParams(needs_layout_passes=False, use_tc_tiling_on_sc=False),
)
def _sc_degree(dst_hbm, ew_hbm, out_hbm, degbuf, didx, ewb):
    c = lax.axis_index("c")
    s = lax.axis_index("s")
    wid = s * NC + c
    zero = jnp.zeros((L,), jnp.float32)

    @pl.loop(0, N // L, unroll=8)
    def _zero(i):
        degbuf[pl.ds(i * L, L)] = zero

    base0 = wid * EPW

    @pl.loop(0, NCHUNK)
    def _chunk(cnk):
        base = base0 + cnk * K
        pltpu.sync_copy(dst_hbm.at[pl.ds(base, K)], didx)
        pltpu.sync_copy(ew_hbm.at[pl.ds(base, K)], ewb)

        @pl.loop(0, K // L, unroll=5)
        def _grp(g):
            dv = didx[pl.ds(g * L, L)]
            ev = ewb[pl.ds(g * L, L)]
            plsc.addupdate_scatter(degbuf, [dv], ev)

    pltpu.sync_copy(degbuf, out_hbm.at[wid])


@functools.partial(
    pl.kernel,
    mesh=_mesh,
    out_type=jax.ShapeDtypeStruct((NC, N, H), jnp.float32),
    scratch_types=[
        pltpu.VMEM_SHARED((N, H), jnp.float32),
        pltpu.VMEM((RPT, H), jnp.float32),
        pltpu.VMEM((K, H), jnp.float32),
        pltpu.VMEM((K,), jnp.int32),
        pltpu.VMEM((1, K), jnp.int32),
        pltpu.VMEM((K,), jnp.float32),
        pltpu.SemaphoreType.DMA,
    ],
    compiler_params=pltpu.CompilerParams(needs_layout_passes=False, use_tc_tiling_on_sc=False),
)
def _sc_propagate(src_hbm, dst_hbm, ew_hbm, hp_hbm, out_hbm,
                  zsh, zbuf, rows, sidx, didx, ewb, sem):
    c = lax.axis_index("c")
    s = lax.axis_index("s")
    wid = s * NC + c
    zero = jnp.zeros((L,), jnp.float32)

    @pl.loop(0, RPT, unroll=4)
    def _zeroloop(r):
        for j in range(H // L):
            zbuf[r, pl.ds(j * L, L)] = zero

    pltpu.sync_copy(zbuf, zsh.at[pl.ds(s * RPT, RPT)])
    plsc.subcore_barrier()

    base0 = wid * EPW

    @pl.loop(0, NCHUNK)
    def _chunk(cnk):
        base = base0 + cnk * K
        pltpu.sync_copy(src_hbm.at[pl.ds(base, K)], sidx)
        pltpu.sync_copy(dst_hbm.at[pl.ds(base, K)], didx.at[0])
        pltpu.sync_copy(ew_hbm.at[pl.ds(base, K)], ewb)
        pltpu.async_copy(hp_hbm.at[sidx], rows, sem).wait()

        @pl.loop(0, K, unroll=4)
        def _scale(e):
            b = plsc.load_gather(ewb, [jnp.full((L,), 0, jnp.int32) + e])
            for j in range(H // L):
                rows[e, pl.ds(j * L, L)] = rows[e, pl.ds(j * L, L)] * b

        pltpu.sync_copy(rows, zsh.at[didx.at[0]], add=True)

    plsc.subcore_barrier()
    pltpu.sync_copy(zsh.at[pl.ds(s * RPT, RPT)],
                    out_hbm.at[c, pl.ds(s * RPT, RPT)])


R = 1024
GRID = (N + R - 1) // R

_ALPHA = 1.6732632423543772848170429916717
_SCALE = 1.0507009873554804934193349852946


def _selu(x):
    return _SCALE * jnp.where(x > 0, x, _ALPHA * (jnp.exp(x) - 1.0))


def _tc1_body(dp_ref, x_ref, w_ref, hp_ref, dinv_ref):
    deg = jnp.sum(dp_ref[...], axis=0) + 1.0
    dinv = jnp.where(deg > 0, lax.rsqrt(deg), 0.0)
    h = jnp.dot(x_ref[...], w_ref[...], preferred_element_type=jnp.float32)
    hp_ref[...] = h * dinv[:, None]
    dinv_ref[...] = dinv


def _tc1(deg_parts, x, gW1):
    return pl.pallas_call(
        _tc1_body,
        grid=(GRID,),
        in_specs=[
            pl.BlockSpec((NW, R), lambda i: (0, i)),
            pl.BlockSpec((R, D_IN), lambda i: (i, 0)),
            pl.BlockSpec((D_IN, H), lambda i: (0, 0)),
        ],
        out_specs=[
            pl.BlockSpec((R, H), lambda i: (i, 0)),
            pl.BlockSpec((R,), lambda i: (i,)),
        ],
        out_shape=[
            jax.ShapeDtypeStruct((N, H), jnp.float32),
            jax.ShapeDtypeStruct((N,), jnp.float32),
        ],
    )(deg_parts, x, gW1)


def _tc2_body(z_ref, hp_ref, dinv_ref, b1_ref, w2_ref, gp_ref):
    z = z_ref[0] + z_ref[1]
    dinv = dinv_ref[...]
    h2 = _selu((z + hp_ref[...]) * dinv[:, None] + b1_ref[...])
    g = jnp.dot(h2, w2_ref[...], preferred_element_type=jnp.float32)
    gp_ref[...] = g * dinv[:, None]


def _tc2(z_parts, hp, dinv, gb1, gW2):
    return pl.pallas_call(
        _tc2_body,
        grid=(GRID,),
        in_specs=[
            pl.BlockSpec((NC, R, H), lambda i: (0, i, 0)),
            pl.BlockSpec((R, H), lambda i: (i, 0)),
            pl.BlockSpec((R,), lambda i: (i,)),
            pl.BlockSpec((1, H), lambda i: (0, 0)),
            pl.BlockSpec((H, H), lambda i: (0, 0)),
        ],
        out_specs=pl.BlockSpec((R, H), lambda i: (i, 0)),
        out_shape=jax.ShapeDtypeStruct((N, H), jnp.float32),
    )(z_parts, hp, dinv, gb1.reshape(1, H), gW2)


def _tc3_body(z_ref, gp_ref, dinv_ref, b2_ref,
              w0_ref, b0_ref, w1_ref, b1_ref, w2_ref, b2m_ref, w3_ref,
              b3_ref, out_ref):
    z = z_ref[0] + z_ref[1]
    dinv = dinv_ref[...]
    h = _selu((z + gp_ref[...]) * dinv[:, None] + b2_ref[...])
    h = _selu(jnp.dot(h, w0_ref[...], preferred_element_type=jnp.float32)
              + b0_ref[...])
    h = _selu(jnp.dot(h, w1_ref[...], preferred_element_type=jnp.float32)
              + b1_ref[...])
    h = _selu(jnp.dot(h, w2_ref[...], preferred_element_type=jnp.float32)
              + b2m_ref[...])
    out_ref[...] = (jnp.dot(h, w3_ref[...], preferred_element_type=jnp.float32)
                    + b3_ref[...])


def _tc3(z_parts, gp, dinv, gb2, lW0, lb0, lW1, lb1, lW2, lb2, lW3, lb3):
    wspec = pl.BlockSpec((H, H), lambda i: (0, 0))
    bspec = pl.BlockSpec((1, H), lambda i: (0, 0))
    return pl.pallas_call(
        _tc3_body,
        grid=(GRID,),
        in_specs=[
            pl.BlockSpec((NC, R, H), lambda i: (0, i, 0)),
            pl.BlockSpec((R, H), lambda i: (i, 0)),
            pl.BlockSpec((R,), lambda i: (i,)),
            bspec, wspec, bspec, wspec, bspec, wspec, bspec,
            pl.BlockSpec((H, D_OUT), lambda i: (0, 0)),
            pl.BlockSpec((1, D_OUT), lambda i: (0, 0)),
        ],
        out_specs=pl.BlockSpec((R, D_OUT), lambda i: (i, 0)),
        out_shape=jax.ShapeDtypeStruct((N, D_OUT), jnp.float32),
    )(z_parts, gp, dinv, gb2.reshape(1, H),
      lW0, lb0.reshape(1, H), lW1, lb1.reshape(1, H), lW2, lb2.reshape(1, H),
      lW3, lb3.reshape(1, D_OUT))


@jax.jit
def kernel(x, edge_index, edge_attrs, gW1, gb1, gW2, gb2,
           lW0, lb0, lW1, lb1, lW2, lb2, lW3, lb3):
    src = edge_index[0]
    dst = edge_index[1]

    deg_parts = _sc_degree(dst, edge_attrs)
    hp, dinv = _tc1(deg_parts, x, gW1)
    z1 = _sc_propagate(src, dst, edge_attrs, hp)
    gp = _tc2(z1, hp, dinv, gb1, gW2)
    z2 = _sc_propagate(src, dst, edge_attrs, gp)
    out = _tc3(z2, gp, dinv, gb2, lW0, lb0, lW1, lb1, lW2, lb2, lW3, lb3)
    return out


# speedup vs baseline: 17.4319x; 1.6604x over previous
import functools
import jax
import jax.numpy as jnp
from jax import lax
from jax.experimental import pallas as pl
from jax.experimental.pallas import tpu as pltpu
from jax.experimental.pallas import tpu_sc as plsc

N = 10000
E = 320000
D_IN = 128
H = 64
D_OUT = 128

NC = 2
NS = 16
NW = NC * NS
L = 16

K = 128
NCHUNK = 80
EPW = K * NCHUNK
EPAD = NW * EPW
RPT = N // NS

_mesh = plsc.VectorSubcoreMesh(
    core_axis_name="c", subcore_axis_name="s", num_cores=NC, num_subcores=NS)

_sc_params = pltpu.CompilerParams(
    needs_layout_passes=False, use_tc_tiling_on_sc=False)


@functools.partial(
    pl.kernel,
    mesh=_mesh,
    out_type=jax.ShapeDtypeStruct((NW, N), jnp.float32),
    scratch_types=[
        pltpu.VMEM((N,), jnp.float32),
        pltpu.VMEM((NCHUNK, K), jnp.int32),
        pltpu.VMEM((NCHUNK, K), jnp.float32),
    ],
    compiler_params=_sc_params,
)
def _sc_degree(dst_hbm, ew_hbm, out_hbm, degbuf, didx, ewa):
    c = lax.axis_index("c")
    s = lax.axis_index("s")
    wid = s * NC + c
    zero = jnp.zeros((L,), jnp.float32)

    pltpu.sync_copy(dst_hbm.at[wid], didx)
    pltpu.sync_copy(ew_hbm.at[wid], ewa)

    @pl.loop(0, N // L, unroll=8)
    def _zero(i):
        degbuf[pl.ds(i * L, L)] = zero

    @pl.loop(0, NCHUNK)
    def _chunk(cnk):
        @pl.loop(0, K // L, unroll=8)
        def _grp(g):
            dv = didx[cnk, pl.ds(g * L, L)]
            ev = ewa[cnk, pl.ds(g * L, L)]
            plsc.addupdate_scatter(degbuf, [dv], ev)

    pltpu.sync_copy(degbuf, out_hbm.at[wid])


@functools.partial(
    pl.kernel,
    mesh=_mesh,
    out_type=jax.ShapeDtypeStruct((NC, N, H), jnp.float32),
    scratch_types=[
        pltpu.VMEM_SHARED((N, H), jnp.float32),
        pltpu.VMEM((RPT, H), jnp.float32),
        pltpu.VMEM((K, H), jnp.float32),
        pltpu.VMEM((K, H), jnp.float32),
        pltpu.VMEM((NCHUNK, K), jnp.int32),
        pltpu.VMEM((NCHUNK, K), jnp.int32),
        pltpu.VMEM((NCHUNK, K), jnp.float32),
        pltpu.SemaphoreType.DMA,
        pltpu.SemaphoreType.DMA,
    ],
    compiler_params=_sc_params,
)
def _sc_propagate(src_hbm, dst_hbm, ew_hbm, hp_hbm, out_hbm,
                  zsh, zbuf, rows0, rows1, sidx, didx, ewa, sem0, sem1):
    c = lax.axis_index("c")
    s = lax.axis_index("s")
    wid = s * NC + c
    zero = jnp.zeros((L,), jnp.float32)

    pltpu.sync_copy(src_hbm.at[wid], sidx)
    pltpu.sync_copy(dst_hbm.at[wid], didx)
    pltpu.sync_copy(ew_hbm.at[wid], ewa)

    pltpu.async_copy(hp_hbm.at[sidx.at[0]], rows0, sem0)
    pltpu.async_copy(hp_hbm.at[sidx.at[1]], rows1, sem1)

    @pl.loop(0, RPT, unroll=8)
    def _zeroloop(r):
        for j in range(H // L):
            zbuf[r, pl.ds(j * L, L)] = zero

    pltpu.sync_copy(zbuf, zsh.at[pl.ds(s * RPT, RPT)])
    plsc.subcore_barrier()

    rows_bufs = (rows0, rows1)
    sems = (sem0, sem1)

    @pl.loop(0, NCHUNK, step=2)
    def _pair(c0):
        for b in range(2):
            cc = c0 + b
            rows = rows_bufs[b]
            sem = sems[b]
            pltpu.make_async_copy(hp_hbm.at[sidx.at[0]], rows, sem).wait()

            idx_c = jnp.full((L,), cc, jnp.int32)

            @pl.loop(0, K, unroll=4)
            def _scale(e):
                w = plsc.load_gather(ewa, [idx_c, jnp.full((L,), e, jnp.int32)])
                for j in range(H // L):
                    rows[e, pl.ds(j * L, L)] = rows[e, pl.ds(j * L, L)] * w

            pltpu.sync_copy(rows, zsh.at[didx.at[cc]], add=True)

            nxt = cc + 2

            @pl.when(nxt < NCHUNK)
            def _prefetch():
                pltpu.async_copy(hp_hbm.at[sidx.at[nxt]], rows, sem)

    plsc.subcore_barrier()
    pltpu.sync_copy(zsh.at[pl.ds(s * RPT, RPT)],
                    out_hbm.at[c, pl.ds(s * RPT, RPT)])


R = 1024
GRID = (N + R - 1) // R

_ALPHA = 1.6732632423543772848170429916717
_SCALE = 1.0507009873554804934193349852946


def _selu(x):
    return _SCALE * jnp.where(x > 0, x, _ALPHA * (jnp.exp(x) - 1.0))


def _tc1_body(dp_ref, x_ref, w_ref, hp_ref, dinv_ref):
    deg = jnp.sum(dp_ref[...], axis=0) + 1.0
    dinv = jnp.where(deg > 0, lax.rsqrt(deg), 0.0)
    h = jnp.dot(x_ref[...], w_ref[...], preferred_element_type=jnp.float32)
    hp_ref[...] = h * dinv[:, None]
    dinv_ref[...] = dinv


def _tc1(deg_parts, x, gW1):
    return pl.pallas_call(
        _tc1_body,
        grid=(GRID,),
        in_specs=[
            pl.BlockSpec((NW, R), lambda i: (0, i)),
            pl.BlockSpec((R, D_IN), lambda i: (i, 0)),
            pl.BlockSpec((D_IN, H), lambda i: (0, 0)),
        ],
        out_specs=[
            pl.BlockSpec((R, H), lambda i: (i, 0)),
            pl.BlockSpec((R,), lambda i: (i,)),
        ],
        out_shape=[
            jax.ShapeDtypeStruct((N, H), jnp.float32),
            jax.ShapeDtypeStruct((N,), jnp.float32),
        ],
    )(deg_parts, x, gW1)


def _tc2_body(z_ref, hp_ref, dinv_ref, b1_ref, w2_ref, gp_ref):
    z = z_ref[0] + z_ref[1]
    dinv = dinv_ref[...]
    h2 = _selu((z + hp_ref[...]) * dinv[:, None] + b1_ref[...])
    g = jnp.dot(h2, w2_ref[...], preferred_element_type=jnp.float32)
    gp_ref[...] = g * dinv[:, None]


def _tc2(z_parts, hp, dinv, gb1, gW2):
    return pl.pallas_call(
        _tc2_body,
        grid=(GRID,),
        in_specs=[
            pl.BlockSpec((NC, R, H), lambda i: (0, i, 0)),
            pl.BlockSpec((R, H), lambda i: (i, 0)),
            pl.BlockSpec((R,), lambda i: (i,)),
            pl.BlockSpec((1, H), lambda i: (0, 0)),
            pl.BlockSpec((H, H), lambda i: (0, 0)),
        ],
        out_specs=pl.BlockSpec((R, H), lambda i: (i, 0)),
        out_shape=jax.ShapeDtypeStruct((N, H), jnp.float32),
    )(z_parts, hp, dinv, gb1.reshape(1, H), gW2)


def _tc3_body(z_ref, gp_ref, dinv_ref, b2_ref,
              w0_ref, b0_ref, w1_ref, b1_ref, w2_ref, b2m_ref, w3_ref,
              b3_ref, out_ref):
    z = z_ref[0] + z_ref[1]
    dinv = dinv_ref[...]
    h = _selu((z + gp_ref[...]) * dinv[:, None] + b2_ref[...])
    h = _selu(jnp.dot(h, w0_ref[...], preferred_element_type=jnp.float32)
              + b0_ref[...])
    h = _selu(jnp.dot(h, w1_ref[...], preferred_element_type=jnp.float32)
              + b1_ref[...])
    h = _selu(jnp.dot(h, w2_ref[...], preferred_element_type=jnp.float32)
              + b2m_ref[...])
    out_ref[...] = (jnp.dot(h, w3_ref[...], preferred_element_type=jnp.float32)
                    + b3_ref[...])


def _tc3(z_parts, gp, dinv, gb2, lW0, lb0, lW1, lb1, lW2, lb2, lW3, lb3):
    wspec = pl.BlockSpec((H, H), lambda i: (0, 0))
    bspec = pl.BlockSpec((1, H), lambda i: (0, 0))
    return pl.pallas_call(
        _tc3_body,
        grid=(GRID,),
        in_specs=[
            pl.BlockSpec((NC, R, H), lambda i: (0, i, 0)),
            pl.BlockSpec((R, H), lambda i: (i, 0)),
            pl.BlockSpec((R,), lambda i: (i,)),
            bspec, wspec, bspec, wspec, bspec, wspec, bspec,
            pl.BlockSpec((H, D_OUT), lambda i: (0, 0)),
            pl.BlockSpec((1, D_OUT), lambda i: (0, 0)),
        ],
        out_specs=pl.BlockSpec((R, D_OUT), lambda i: (i, 0)),
        out_shape=jax.ShapeDtypeStruct((N, D_OUT), jnp.float32),
    )(z_parts, gp, dinv, gb2.reshape(1, H),
      lW0, lb0.reshape(1, H), lW1, lb1.reshape(1, H), lW2, lb2.reshape(1, H),
      lW3, lb3.reshape(1, D_OUT))


@jax.jit
def kernel(x, edge_index, edge_attrs, gW1, gb1, gW2, gb2,
           lW0, lb0, lW1, lb1, lW2, lb2, lW3, lb3):
    src = edge_index[0]
    dst = edge_index[1]

    pad = EPAD - E
    srcp = jnp.concatenate(
        [src, jnp.zeros((pad,), jnp.int32)]).reshape(NW, NCHUNK, K)
    dstp = jnp.concatenate(
        [dst, jnp.zeros((pad,), jnp.int32)]).reshape(NW, NCHUNK, K)
    ewp = jnp.concatenate(
        [edge_attrs, jnp.zeros((pad,), jnp.float32)]).reshape(NW, NCHUNK, K)

    deg_parts = _sc_degree(dstp, ewp)
    hp, dinv = _tc1(deg_parts, x, gW1)
    z1 = _sc_propagate(srcp, dstp, ewp, hp)
    gp = _tc2(z1, hp, dinv, gb1, gW2)
    z2 = _sc_propagate(srcp, dstp, ewp, gp)
    out = _tc3(z2, gp, dinv, gb2, lW0, lb0, lW1, lb1, lW2, lb2, lW3, lb3)
    return out


# speedup vs baseline: 17.4726x; 1.0023x over previous
import functools
import jax
import jax.numpy as jnp
from jax import lax
from jax.experimental import pallas as pl
from jax.experimental.pallas import tpu as pltpu
from jax.experimental.pallas import tpu_sc as plsc

N = 10000
E = 320000
D_IN = 128
H = 64
D_OUT = 128

NC = 2
NS = 16
NW = NC * NS
L = 16

K = 128
NCHUNK = 80
EPW = K * NCHUNK
EPAD = NW * EPW
RPT = N // NS

_mesh = plsc.VectorSubcoreMesh(
    core_axis_name="c", subcore_axis_name="s", num_cores=NC, num_subcores=NS)

_sc_params = pltpu.CompilerParams(
    needs_layout_passes=False, use_tc_tiling_on_sc=False)


@functools.partial(
    pl.kernel,
    mesh=_mesh,
    out_type=jax.ShapeDtypeStruct((NW, N), jnp.float32),
    scratch_types=[
        pltpu.VMEM((N,), jnp.float32),
        pltpu.VMEM((NCHUNK, K), jnp.int32),
        pltpu.VMEM((NCHUNK, K), jnp.float32),
    ],
    compiler_params=_sc_params,
)
def _sc_degree(dst_hbm, ew_hbm, out_hbm, degbuf, didx, ewa):
    c = lax.axis_index("c")
    s = lax.axis_index("s")
    wid = s * NC + c
    zero = jnp.zeros((L,), jnp.float32)

    pltpu.sync_copy(dst_hbm.at[wid], didx)
    pltpu.sync_copy(ew_hbm.at[wid], ewa)

    @pl.loop(0, N // L, unroll=8)
    def _zero(i):
        degbuf[pl.ds(i * L, L)] = zero

    @pl.loop(0, NCHUNK)
    def _chunk(cnk):
        @pl.loop(0, K // L, unroll=8)
        def _grp(g):
            dv = didx[cnk, pl.ds(g * L, L)]
            ev = ewa[cnk, pl.ds(g * L, L)]
            plsc.addupdate_scatter(degbuf, [dv], ev)

    pltpu.sync_copy(degbuf, out_hbm.at[wid])


@functools.partial(
    pl.kernel,
    mesh=_mesh,
    out_type=jax.ShapeDtypeStruct((NC, N, H), jnp.float32),
    scratch_types=[
        pltpu.VMEM_SHARED((N, H), jnp.float32),
        [pltpu.VMEM((K, H), jnp.float32)] * 4,
        pltpu.VMEM((NCHUNK, K), jnp.int32),
        pltpu.VMEM((NCHUNK, K), jnp.int32),
        pltpu.VMEM((NCHUNK, K), jnp.float32),
        [pltpu.SemaphoreType.DMA] * 4,
        [pltpu.SemaphoreType.DMA] * 4,
    ],
    compiler_params=_sc_params,
)
def _sc_propagate(src_hbm, dst_hbm, ew_hbm, hp_hbm, out_hbm,
                  zsh, rows_bufs, sidx, didx, ewa, gsems, ssems):
    c = lax.axis_index("c")
    s = lax.axis_index("s")
    wid = s * NC + c
    zero = jnp.zeros((L,), jnp.float32)

    pltpu.sync_copy(src_hbm.at[wid], sidx)
    pltpu.sync_copy(dst_hbm.at[wid], didx)
    pltpu.sync_copy(ew_hbm.at[wid], ewa)

    for b in range(3):
        pltpu.async_copy(hp_hbm.at[sidx.at[b]], rows_bufs[b], gsems[b])

    zb = rows_bufs[3]

    @pl.loop(0, K, unroll=8)
    def _zeroloop(r):
        for j in range(H // L):
            zb[r, pl.ds(j * L, L)] = zero

    for k in range(5):
        pltpu.sync_copy(zb.at[pl.ds(0, RPT // 5)],
                        zsh.at[pl.ds(s * RPT + k * (RPT // 5), RPT // 5)])

    plsc.subcore_barrier()

    @pl.loop(0, NCHUNK, step=4)
    def _quad(c0):
        for b in range(4):
            cc = c0 + b
            rows = rows_bufs[b]
            pltpu.make_async_copy(hp_hbm.at[sidx.at[0]], rows,
                                  gsems[b]).wait()

            idx_c = jnp.full((L,), cc, jnp.int32)

            @pl.loop(0, K, unroll=8)
            def _scale(e):
                w = plsc.load_gather(ewa, [idx_c, jnp.full((L,), e, jnp.int32)])
                for j in range(H // L):
                    rows[e, pl.ds(j * L, L)] = rows[e, pl.ds(j * L, L)] * w

            pltpu.sync_copy(rows, zsh.at[didx.at[cc]], add=True)

            nxt = cc + 3
            bn = (b + 3) % 4

            @pl.when(nxt < NCHUNK)
            def _prefetch():
                pltpu.async_copy(hp_hbm.at[sidx.at[nxt]], rows_bufs[bn],
                                 gsems[bn])

    plsc.subcore_barrier()
    pltpu.sync_copy(zsh.at[pl.ds(s * RPT, RPT)],
                    out_hbm.at[c, pl.ds(s * RPT, RPT)])


R = 1024
GRID = (N + R - 1) // R

_ALPHA = 1.6732632423543772848170429916717
_SCALE = 1.0507009873554804934193349852946


def _selu(x):
    return _SCALE * jnp.where(x > 0, x, _ALPHA * (jnp.exp(x) - 1.0))


def _tc1_body(dp_ref, x_ref, w_ref, hp_ref, dinv_ref):
    deg = jnp.sum(dp_ref[...], axis=0) + 1.0
    dinv = jnp.where(deg > 0, lax.rsqrt(deg), 0.0)
    h = jnp.dot(x_ref[...], w_ref[...], preferred_element_type=jnp.float32)
    hp_ref[...] = h * dinv[:, None]
    dinv_ref[...] = dinv


def _tc1(deg_parts, x, gW1):
    return pl.pallas_call(
        _tc1_body,
        grid=(GRID,),
        in_specs=[
            pl.BlockSpec((NW, R), lambda i: (0, i)),
            pl.BlockSpec((R, D_IN), lambda i: (i, 0)),
            pl.BlockSpec((D_IN, H), lambda i: (0, 0)),
        ],
        out_specs=[
            pl.BlockSpec((R, H), lambda i: (i, 0)),
            pl.BlockSpec((R,), lambda i: (i,)),
        ],
        out_shape=[
            jax.ShapeDtypeStruct((N, H), jnp.float32),
            jax.ShapeDtypeStruct((N,), jnp.float32),
        ],
    )(deg_parts, x, gW1)


def _tc2_body(z_ref, hp_ref, dinv_ref, b1_ref, w2_ref, gp_ref):
    z = z_ref[0] + z_ref[1]
    dinv = dinv_ref[...]
    h2 = _selu((z + hp_ref[...]) * dinv[:, None] + b1_ref[...])
    g = jnp.dot(h2, w2_ref[...], preferred_element_type=jnp.float32)
    gp_ref[...] = g * dinv[:, None]


def _tc2(z_parts, hp, dinv, gb1, gW2):
    return pl.pallas_call(
        _tc2_body,
        grid=(GRID,),
        in_specs=[
            pl.BlockSpec((NC, R, H), lambda i: (0, i, 0)),
            pl.BlockSpec((R, H), lambda i: (i, 0)),
            pl.BlockSpec((R,), lambda i: (i,)),
            pl.BlockSpec((1, H), lambda i: (0, 0)),
            pl.BlockSpec((H, H), lambda i: (0, 0)),
        ],
        out_specs=pl.BlockSpec((R, H), lambda i: (i, 0)),
        out_shape=jax.ShapeDtypeStruct((N, H), jnp.float32),
    )(z_parts, hp, dinv, gb1.reshape(1, H), gW2)


def _tc3_body(z_ref, gp_ref, dinv_ref, b2_ref,
              w0_ref, b0_ref, w1_ref, b1_ref, w2_ref, b2m_ref, w3_ref,
              b3_ref, out_ref):
    z = z_ref[0] + z_ref[1]
    dinv = dinv_ref[...]
    h = _selu((z + gp_ref[...]) * dinv[:, None] + b2_ref[...])
    h = _selu(jnp.dot(h, w0_ref[...], preferred_element_type=jnp.float32)
              + b0_ref[...])
    h = _selu(jnp.dot(h, w1_ref[...], preferred_element_type=jnp.float32)
              + b1_ref[...])
    h = _selu(jnp.dot(h, w2_ref[...], preferred_element_type=jnp.float32)
              + b2m_ref[...])
    out_ref[...] = (jnp.dot(h, w3_ref[...], preferred_element_type=jnp.float32)
                    + b3_ref[...])


def _tc3(z_parts, gp, dinv, gb2, lW0, lb0, lW1, lb1, lW2, lb2, lW3, lb3):
    wspec = pl.BlockSpec((H, H), lambda i: (0, 0))
    bspec = pl.BlockSpec((1, H), lambda i: (0, 0))
    return pl.pallas_call(
        _tc3_body,
        grid=(GRID,),
        in_specs=[
            pl.BlockSpec((NC, R, H), lambda i: (0, i, 0)),
            pl.BlockSpec((R, H), lambda i: (i, 0)),
            pl.BlockSpec((R,), lambda i: (i,)),
            bspec, wspec, bspec, wspec, bspec, wspec, bspec,
            pl.BlockSpec((H, D_OUT), lambda i: (0, 0)),
            pl.BlockSpec((1, D_OUT), lambda i: (0, 0)),
        ],
        out_specs=pl.BlockSpec((R, D_OUT), lambda i: (i, 0)),
        out_shape=jax.ShapeDtypeStruct((N, D_OUT), jnp.float32),
    )(z_parts, gp, dinv, gb2.reshape(1, H),
      lW0, lb0.reshape(1, H), lW1, lb1.reshape(1, H), lW2, lb2.reshape(1, H),
      lW3, lb3.reshape(1, D_OUT))


@jax.jit
def kernel(x, edge_index, edge_attrs, gW1, gb1, gW2, gb2,
           lW0, lb0, lW1, lb1, lW2, lb2, lW3, lb3):
    src = edge_index[0]
    dst = edge_index[1]

    pad = EPAD - E
    srcp = jnp.concatenate(
        [src, jnp.zeros((pad,), jnp.int32)]).reshape(NW, NCHUNK, K)
    dstp = jnp.concatenate(
        [dst, jnp.zeros((pad,), jnp.int32)]).reshape(NW, NCHUNK, K)
    ewp = jnp.concatenate(
        [edge_attrs, jnp.zeros((pad,), jnp.float32)]).reshape(NW, NCHUNK, K)

    deg_parts = _sc_degree(dstp, ewp)
    hp, dinv = _tc1(deg_parts, x, gW1)
    z1 = _sc_propagate(srcp, dstp, ewp, hp)
    gp = _tc2(z1, hp, dinv, gb1, gW2)
    z2 = _sc_propagate(srcp, dstp, ewp, gp)
    out = _tc3(z2, gp, dinv, gb2, lW0, lb0, lW1, lb1, lW2, lb2, lW3, lb3)
    return out


# speedup vs baseline: 17.6617x; 1.0108x over previous
import functools
import jax
import jax.numpy as jnp
from jax import lax
from jax.experimental import pallas as pl
from jax.experimental.pallas import tpu as pltpu
from jax.experimental.pallas import tpu_sc as plsc

N = 10000
E = 320000
D_IN = 128
H = 64
D_OUT = 128

NC = 2
NS = 16
NW = NC * NS
L = 16

K = 128
NCHUNK = 80
EPW = K * NCHUNK
EPAD = NW * EPW
RPT = N // NS

_mesh = plsc.VectorSubcoreMesh(
    core_axis_name="c", subcore_axis_name="s", num_cores=NC, num_subcores=NS)

_sc_params = pltpu.CompilerParams(
    needs_layout_passes=False, use_tc_tiling_on_sc=False)


@functools.partial(
    pl.kernel,
    mesh=_mesh,
    out_type=jax.ShapeDtypeStruct((NW, N), jnp.float32),
    scratch_types=[
        pltpu.VMEM((N,), jnp.float32),
        pltpu.VMEM((NCHUNK, K), jnp.int32),
        pltpu.VMEM((NCHUNK, K), jnp.float32),
    ],
    compiler_params=_sc_params,
)
def _sc_degree(dst_hbm, ew_hbm, out_hbm, degbuf, didx, ewa):
    c = lax.axis_index("c")
    s = lax.axis_index("s")
    wid = s * NC + c
    zero = jnp.zeros((L,), jnp.float32)

    pltpu.sync_copy(dst_hbm.at[wid], didx)
    pltpu.sync_copy(ew_hbm.at[wid], ewa)

    @pl.loop(0, N // L, unroll=8)
    def _zero(i):
        degbuf[pl.ds(i * L, L)] = zero

    @pl.loop(0, NCHUNK)
    def _chunk(cnk):
        @pl.loop(0, K // L, unroll=8)
        def _grp(g):
            dv = didx[cnk, pl.ds(g * L, L)]
            ev = ewa[cnk, pl.ds(g * L, L)]
            plsc.addupdate_scatter(degbuf, [dv], ev)

    pltpu.sync_copy(degbuf, out_hbm.at[wid])


@functools.partial(
    pl.kernel,
    mesh=_mesh,
    out_type=jax.ShapeDtypeStruct((NC, N, H), jnp.float32),
    scratch_types=[
        pltpu.VMEM_SHARED((N, H), jnp.float32),
        [pltpu.VMEM((K, H), jnp.float32)] * 4,
        pltpu.VMEM((NCHUNK, K), jnp.int32),
        pltpu.VMEM((NCHUNK, K), jnp.int32),
        pltpu.VMEM((NCHUNK, K), jnp.float32),
        [pltpu.SemaphoreType.DMA] * 4,
        [pltpu.SemaphoreType.DMA] * 4,
    ],
    compiler_params=_sc_params,
)
def _sc_propagate(src_hbm, dst_hbm, ew_hbm, hp_hbm, out_hbm,
                  zsh, rows_bufs, sidx, didx, ewa, gsems, ssems):
    c = lax.axis_index("c")
    s = lax.axis_index("s")
    wid = s * NC + c
    zero = jnp.zeros((L,), jnp.float32)

    pltpu.sync_copy(src_hbm.at[wid], sidx)
    pltpu.sync_copy(dst_hbm.at[wid], didx)
    pltpu.sync_copy(ew_hbm.at[wid], ewa)

    for b in range(3):
        pltpu.async_copy(hp_hbm.at[sidx.at[b]], rows_bufs[b], gsems[b])

    zb = rows_bufs[3]

    @pl.loop(0, K, unroll=8)
    def _zeroloop(r):
        for j in range(H // L):
            zb[r, pl.ds(j * L, L)] = zero

    for k in range(5):
        pltpu.sync_copy(zb.at[pl.ds(0, RPT // 5)],
                        zsh.at[pl.ds(s * RPT + k * (RPT // 5), RPT // 5)])

    plsc.subcore_barrier()

    @pl.loop(0, NCHUNK, step=4)
    def _quad(c0):
        for b in range(4):
            cc = c0 + b
            rows = rows_bufs[b]
            pltpu.make_async_copy(hp_hbm.at[sidx.at[0]], rows,
                                  gsems[b]).wait()

            idx_c = jnp.full((L,), cc, jnp.int32)

            @pl.loop(0, K, unroll=8)
            def _scale(e):
                w = plsc.load_gather(ewa, [idx_c, jnp.full((L,), e, jnp.int32)])
                for j in range(H // L):
                    rows[e, pl.ds(j * L, L)] = rows[e, pl.ds(j * L, L)] * w

            bn = (b + 3) % 4

            @pl.when(cc >= 1)
            def _retire():
                pltpu.make_async_copy(hp_hbm.at[sidx.at[0]], rows_bufs[bn],
                                      ssems[bn]).wait()

            pltpu.async_copy(rows, zsh.at[didx.at[cc]], ssems[b], add=True)

            nxt = cc + 3

            @pl.when(nxt < NCHUNK)
            def _prefetch():
                pltpu.async_copy(hp_hbm.at[sidx.at[nxt]], rows_bufs[bn],
                                 gsems[bn])

    pltpu.make_async_copy(hp_hbm.at[sidx.at[0]], rows_bufs[3], ssems[3]).wait()

    plsc.subcore_barrier()
    pltpu.sync_copy(zsh.at[pl.ds(s * RPT, RPT)],
                    out_hbm.at[c, pl.ds(s * RPT, RPT)])


R = 1024
GRID = (N + R - 1) // R

_ALPHA = 1.6732632423543772848170429916717
_SCALE = 1.0507009873554804934193349852946


def _selu(x):
    return _SCALE * jnp.where(x > 0, x, _ALPHA * (jnp.exp(x) - 1.0))


def _tc1_body(dp_ref, x_ref, w_ref, hp_ref, dinv_ref):
    deg = jnp.sum(dp_ref[...], axis=0) + 1.0
    dinv = jnp.where(deg > 0, lax.rsqrt(deg), 0.0)
    h = jnp.dot(x_ref[...], w_ref[...], preferred_element_type=jnp.float32)
    hp_ref[...] = h * dinv[:, None]
    dinv_ref[...] = dinv


def _tc1(deg_parts, x, gW1):
    return pl.pallas_call(
        _tc1_body,
        grid=(GRID,),
        in_specs=[
            pl.BlockSpec((NW, R), lambda i: (0, i)),
            pl.BlockSpec((R, D_IN), lambda i: (i, 0)),
            pl.BlockSpec((D_IN, H), lambda i: (0, 0)),
        ],
        out_specs=[
            pl.BlockSpec((R, H), lambda i: (i, 0)),
            pl.BlockSpec((R,), lambda i: (i,)),
        ],
        out_shape=[
            jax.ShapeDtypeStruct((N, H), jnp.float32),
            jax.ShapeDtypeStruct((N,), jnp.float32),
        ],
    )(deg_parts, x, gW1)


def _tc2_body(z_ref, hp_ref, dinv_ref, b1_ref, w2_ref, gp_ref):
    z = z_ref[0] + z_ref[1]
    dinv = dinv_ref[...]
    h2 = _selu((z + hp_ref[...]) * dinv[:, None] + b1_ref[...])
    g = jnp.dot(h2, w2_ref[...], preferred_element_type=jnp.float32)
    gp_ref[...] = g * dinv[:, None]


def _tc2(z_parts, hp, dinv, gb1, gW2):
    return pl.pallas_call(
        _tc2_body,
        grid=(GRID,),
        in_specs=[
            pl.BlockSpec((NC, R, H), lambda i: (0, i, 0)),
            pl.BlockSpec((R, H), lambda i: (i, 0)),
            pl.BlockSpec((R,), lambda i: (i,)),
            pl.BlockSpec((1, H), lambda i: (0, 0)),
            pl.BlockSpec((H, H), lambda i: (0, 0)),
        ],
        out_specs=pl.BlockSpec((R, H), lambda i: (i, 0)),
        out_shape=jax.ShapeDtypeStruct((N, H), jnp.float32),
    )(z_parts, hp, dinv, gb1.reshape(1, H), gW2)


def _tc3_body(z_ref, gp_ref, dinv_ref, b2_ref,
              w0_ref, b0_ref, w1_ref, b1_ref, w2_ref, b2m_ref, w3_ref,
              b3_ref, out_ref):
    z = z_ref[0] + z_ref[1]
    dinv = dinv_ref[...]
    h = _selu((z + gp_ref[...]) * dinv[:, None] + b2_ref[...])
    h = _selu(jnp.dot(h, w0_ref[...], preferred_element_type=jnp.float32)
              + b0_ref[...])
    h = _selu(jnp.dot(h, w1_ref[...], preferred_element_type=jnp.float32)
              + b1_ref[...])
    h = _selu(jnp.dot(h, w2_ref[...], preferred_element_type=jnp.float32)
              + b2m_ref[...])
    out_ref[...] = (jnp.dot(h, w3_ref[...], preferred_element_type=jnp.float32)
                    + b3_ref[...])


def _tc3(z_parts, gp, dinv, gb2, lW0, lb0, lW1, lb1, lW2, lb2, lW3, lb3):
    wspec = pl.BlockSpec((H, H), lambda i: (0, 0))
    bspec = pl.BlockSpec((1, H), lambda i: (0, 0))
    return pl.pallas_call(
        _tc3_body,
        grid=(GRID,),
        in_specs=[
            pl.BlockSpec((NC, R, H), lambda i: (0, i, 0)),
            pl.BlockSpec((R, H), lambda i: (i, 0)),
            pl.BlockSpec((R,), lambda i: (i,)),
            bspec, wspec, bspec, wspec, bspec, wspec, bspec,
            pl.BlockSpec((H, D_OUT), lambda i: (0, 0)),
            pl.BlockSpec((1, D_OUT), lambda i: (0, 0)),
        ],
        out_specs=pl.BlockSpec((R, D_OUT), lambda i: (i, 0)),
        out_shape=jax.ShapeDtypeStruct((N, D_OUT), jnp.float32),
    )(z_parts, gp, dinv, gb2.reshape(1, H),
      lW0, lb0.reshape(1, H), lW1, lb1.reshape(1, H), lW2, lb2.reshape(1, H),
      lW3, lb3.reshape(1, D_OUT))


@jax.jit
def kernel(x, edge_index, edge_attrs, gW1, gb1, gW2, gb2,
           lW0, lb0, lW1, lb1, lW2, lb2, lW3, lb3):
    src = edge_index[0]
    dst = edge_index[1]

    pad = EPAD - E
    srcp = jnp.concatenate(
        [src, jnp.zeros((pad,), jnp.int32)]).reshape(NW, NCHUNK, K)
    dstp = jnp.concatenate(
        [dst, jnp.zeros((pad,), jnp.int32)]).reshape(NW, NCHUNK, K)
    ewp = jnp.concatenate(
        [edge_attrs, jnp.zeros((pad,), jnp.float32)]).reshape(NW, NCHUNK, K)

    deg_parts = _sc_degree(dstp, ewp)
    hp, dinv = _tc1(deg_parts, x, gW1)
    z1 = _sc_propagate(srcp, dstp, ewp, hp)
    gp = _tc2(z1, hp, dinv, gb1, gW2)
    z2 = _sc_propagate(srcp, dstp, ewp, gp)
    out = _tc3(z2, gp, dinv, gb2, lW0, lb0, lW1, lb1, lW2, lb2, lW3, lb3)
    return out


# speedup vs baseline: 23.5954x; 1.3360x over previous
import functools
import numpy as np
import jax
import jax.numpy as jnp
from jax import lax
from jax.experimental import pallas as pl
from jax.experimental.pallas import tpu as pltpu
from jax.experimental.pallas import tpu_sc as plsc

N = 10000
E = 320000
D_IN = 128
H = 64
D_OUT = 128

NC = 2
NS = 16
NW = NC * NS
L = 16

K = 128
NCHUNK = 80
EPW = K * NCHUNK
EPAD = NW * EPW
RPT = N // NS

_mesh = plsc.VectorSubcoreMesh(
    core_axis_name="c", subcore_axis_name="s", num_cores=NC, num_subcores=NS)

_sc_params = pltpu.CompilerParams(
    needs_layout_passes=False, use_tc_tiling_on_sc=False)


@functools.partial(
    pl.kernel,
    mesh=_mesh,
    out_type=jax.ShapeDtypeStruct((NW, N), jnp.float32),
    scratch_types=[
        pltpu.VMEM((N,), jnp.float32),
        pltpu.VMEM((NCHUNK, K), jnp.int32),
        pltpu.VMEM((NCHUNK, K), jnp.float32),
    ],
    compiler_params=_sc_params,
)
def _sc_degree(dst_hbm, ew_hbm, out_hbm, degbuf, didx, ewa):
    c = lax.axis_index("c")
    s = lax.axis_index("s")
    wid = s * NC + c
    zero = jnp.zeros((L,), jnp.float32)

    pltpu.sync_copy(dst_hbm.at[wid], didx)
    pltpu.sync_copy(ew_hbm.at[wid], ewa)

    @pl.loop(0, N // L, unroll=8)
    def _zero(i):
        degbuf[pl.ds(i * L, L)] = zero

    @pl.loop(0, NCHUNK)
    def _chunk(cnk):
        @pl.loop(0, K // L, unroll=8)
        def _grp(g):
            dv = didx[cnk, pl.ds(g * L, L)]
            ev = ewa[cnk, pl.ds(g * L, L)]
            plsc.addupdate_scatter(degbuf, [dv], ev)

    pltpu.sync_copy(degbuf, out_hbm.at[wid])


@functools.partial(
    pl.kernel,
    mesh=_mesh,
    out_type=jax.ShapeDtypeStruct((NC, N, H), jnp.float32),
    scratch_types=[
        pltpu.VMEM_SHARED((N, H), jnp.float32),
        [pltpu.VMEM((K, H // 2), jnp.float32)] * 4,
        [pltpu.VMEM((K, H), jnp.float32)] * 2,
        pltpu.VMEM((NCHUNK, K), jnp.int32),
        pltpu.VMEM((NCHUNK, K), jnp.int32),
        pltpu.VMEM((NCHUNK, K), jnp.float32),
        [pltpu.SemaphoreType.DMA] * 4,
        [pltpu.SemaphoreType.DMA] * 2,
    ],
    compiler_params=_sc_params,
)
def _sc_propagate(src_hbm, dst_hbm, ew_hbm, hp_hbm, out_hbm,
                  zsh, rows_bufs, fbufs, sidx, didx, ewa, gsems, ssems):
    c = lax.axis_index("c")
    s = lax.axis_index("s")
    wid = s * NC + c
    zero = jnp.zeros((L,), jnp.float32)

    pltpu.sync_copy(src_hbm.at[wid], sidx)
    pltpu.sync_copy(dst_hbm.at[wid], didx)
    pltpu.sync_copy(ew_hbm.at[wid], ewa)

    for b in range(3):
        pltpu.async_copy(hp_hbm.at[sidx.at[b]], rows_bufs[b], gsems[b])

    zb = fbufs[1]

    @pl.loop(0, K, unroll=8)
    def _zeroloop(r):
        for j in range(H // L):
            zb[r, pl.ds(j * L, L)] = zero

    for k in range(5):
        pltpu.sync_copy(zb.at[pl.ds(0, RPT // 5)],
                        zsh.at[pl.ds(s * RPT + k * (RPT // 5), RPT // 5)])

    plsc.subcore_barrier()

    @pl.loop(0, NCHUNK, step=4)
    def _quad(c0):
        for b in range(4):
            cc = c0 + b
            rows = rows_bufs[b]
            fb = fbufs[b % 2]
            pltpu.make_async_copy(hp_hbm.at[sidx.at[0]], rows,
                                  gsems[b]).wait()

            @pl.when(cc >= 2)
            def _retire():
                pltpu.make_async_copy(out_hbm.at[c, pl.ds(0, K)], fb,
                                      ssems[b % 2]).wait()

            idx_c = jnp.full((L,), cc, jnp.int32)

            @pl.loop(0, K, unroll=8)
            def _scale(e):
                w = plsc.load_gather(ewa, [idx_c, jnp.full((L,), e, jnp.int32)])
                for g in range(H // 32):
                    pw = rows[e, pl.ds(g * L, L)]
                    pb = plsc.bitcast(pw, jnp.bfloat16)
                    va, vb = plsc.unpack(pb, format=plsc.PackFormat.INTERLEAVED)
                    fb[e, pl.ds(g * 32, L)] = va * w
                    fb[e, pl.ds(g * 32 + L, L)] = vb * w

            pltpu.async_copy(fb, zsh.at[didx.at[cc]], ssems[b % 2], add=True)

            bn = (b + 3) % 4
            nxt = cc + 3

            @pl.when(nxt < NCHUNK)
            def _prefetch():
                pltpu.async_copy(hp_hbm.at[sidx.at[nxt]], rows_bufs[bn],
                                 gsems[bn])

    for p in range(2):
        pltpu.make_async_copy(out_hbm.at[c, pl.ds(0, K)], fbufs[p],
                              ssems[p]).wait()

    plsc.subcore_barrier()
    pltpu.sync_copy(zsh.at[pl.ds(s * RPT, RPT)],
                    out_hbm.at[c, pl.ds(s * RPT, RPT)])


R = 1024
GRID = (N + R - 1) // R

_ALPHA = 1.6732632423543772848170429916717
_SCALE = 1.0507009873554804934193349852946


def _selu(x):
    return _SCALE * jnp.where(x > 0, x, _ALPHA * (jnp.exp(x) - 1.0))


def _tc1_body(dp_ref, x_ref, w_ref, hp_ref, dinv_ref):
    deg = jnp.sum(dp_ref[...], axis=0) + 1.0
    dinv = jnp.where(deg > 0, lax.rsqrt(deg), 0.0)
    h = jnp.dot(x_ref[...], w_ref[...], preferred_element_type=jnp.float32)
    hp_ref[...] = h * dinv[:, None]
    dinv_ref[...] = dinv


def _tc1(deg_parts, x, gW1):
    return pl.pallas_call(
        _tc1_body,
        grid=(GRID,),
        in_specs=[
            pl.BlockSpec((NW, R), lambda i: (0, i)),
            pl.BlockSpec((R, D_IN), lambda i: (i, 0)),
            pl.BlockSpec((D_IN, H), lambda i: (0, 0)),
        ],
        out_specs=[
            pl.BlockSpec((R, H), lambda i: (i, 0)),
            pl.BlockSpec((R,), lambda i: (i,)),
        ],
        out_shape=[
            jax.ShapeDtypeStruct((N, H), jnp.float32),
            jax.ShapeDtypeStruct((N,), jnp.float32),
        ],
    )(deg_parts, x, gW1)


def _tc2_body(z_ref, hp_ref, dinv_ref, b1_ref, w2_ref, gp_ref):
    z = z_ref[0] + z_ref[1]
    dinv = dinv_ref[...]
    h2 = _selu((z + hp_ref[...]) * dinv[:, None] + b1_ref[...])
    g = jnp.dot(h2, w2_ref[...], preferred_element_type=jnp.float32)
    gp_ref[...] = g * dinv[:, None]


def _tc2(z_parts, hp, dinv, gb1, gW2):
    return pl.pallas_call(
        _tc2_body,
        grid=(GRID,),
        in_specs=[
            pl.BlockSpec((NC, R, H), lambda i: (0, i, 0)),
            pl.BlockSpec((R, H), lambda i: (i, 0)),
            pl.BlockSpec((R,), lambda i: (i,)),
            pl.BlockSpec((1, H), lambda i: (0, 0)),
            pl.BlockSpec((H, H), lambda i: (0, 0)),
        ],
        out_specs=pl.BlockSpec((R, H), lambda i: (i, 0)),
        out_shape=jax.ShapeDtypeStruct((N, H), jnp.float32),
    )(z_parts, hp, dinv, gb1.reshape(1, H), gW2)


def _tc3_body(z_ref, gp_ref, dinv_ref, b2_ref,
              w0_ref, b0_ref, w1_ref, b1_ref, w2_ref, b2m_ref, w3_ref,
              b3_ref, out_ref):
    z = z_ref[0] + z_ref[1]
    dinv = dinv_ref[...]
    h = _selu((z + gp_ref[...]) * dinv[:, None] + b2_ref[...])
    h = _selu(jnp.dot(h, w0_ref[...], preferred_element_type=jnp.float32)
              + b0_ref[...])
    h = _selu(jnp.dot(h, w1_ref[...], preferred_element_type=jnp.float32)
              + b1_ref[...])
    h = _selu(jnp.dot(h, w2_ref[...], preferred_element_type=jnp.float32)
              + b2m_ref[...])
    out_ref[...] = (jnp.dot(h, w3_ref[...], preferred_element_type=jnp.float32)
                    + b3_ref[...])


def _tc3(z_parts, gp, dinv, gb2, lW0, lb0, lW1, lb1, lW2, lb2, lW3, lb3):
    wspec = pl.BlockSpec((H, H), lambda i: (0, 0))
    bspec = pl.BlockSpec((1, H), lambda i: (0, 0))
    return pl.pallas_call(
        _tc3_body,
        grid=(GRID,),
        in_specs=[
            pl.BlockSpec((NC, R, H), lambda i: (0, i, 0)),
            pl.BlockSpec((R, H), lambda i: (i, 0)),
            pl.BlockSpec((R,), lambda i: (i,)),
            bspec, wspec, bspec, wspec, bspec, wspec, bspec,
            pl.BlockSpec((H, D_OUT), lambda i: (0, 0)),
            pl.BlockSpec((1, D_OUT), lambda i: (0, 0)),
        ],
        out_specs=pl.BlockSpec((R, D_OUT), lambda i: (i, 0)),
        out_shape=jax.ShapeDtypeStruct((N, D_OUT), jnp.float32),
    )(z_parts, gp, dinv, gb2.reshape(1, H),
      lW0, lb0.reshape(1, H), lW1, lb1.reshape(1, H), lW2, lb2.reshape(1, H),
      lW3, lb3.reshape(1, D_OUT))


_PERM = np.array([32 * g + (i // 2) + 16 * (i % 2)
                  for g in range(2) for i in range(32)], dtype=np.int32)


def _pack_bf16(a):
    p = a.astype(jnp.bfloat16)[:, _PERM].reshape(N, H // 2, 2)
    return jax.lax.bitcast_convert_type(p, jnp.float32)


@jax.jit
def kernel(x, edge_index, edge_attrs, gW1, gb1, gW2, gb2,
           lW0, lb0, lW1, lb1, lW2, lb2, lW3, lb3):
    src = edge_index[0]
    dst = edge_index[1]

    pad = EPAD - E
    srcp = jnp.concatenate(
        [src, jnp.zeros((pad,), jnp.int32)]).reshape(NW, NCHUNK, K)
    dstp = jnp.concatenate(
        [dst, jnp.zeros((pad,), jnp.int32)]).reshape(NW, NCHUNK, K)
    ewp = jnp.concatenate(
        [edge_attrs, jnp.zeros((pad,), jnp.float32)]).reshape(NW, NCHUNK, K)

    deg_parts = _sc_degree(dstp, ewp)
    hp, dinv = _tc1(deg_parts, x, gW1)
    z1 = _sc_propagate(srcp, dstp, ewp, _pack_bf16(hp))
    gp = _tc2(z1, hp, dinv, gb1, gW2)
    z2 = _sc_propagate(srcp, dstp, ewp, _pack_bf16(gp))
    out = _tc3(z2, gp, dinv, gb2, lW0, lb0, lW1, lb1, lW2, lb2, lW3, lb3)
    return out


# speedup vs baseline: 23.8485x; 1.0107x over previous
import functools
import numpy as np
import jax
import jax.numpy as jnp
from jax import lax
from jax.experimental import pallas as pl
from jax.experimental.pallas import tpu as pltpu
from jax.experimental.pallas import tpu_sc as plsc

N = 10000
E = 320000
D_IN = 128
H = 64
D_OUT = 128

NC = 2
NS = 16
NW = NC * NS
L = 16

K = 128
NCHUNK = 80
EPW = K * NCHUNK
EPAD = NW * EPW
RPT = N // NS

_mesh = plsc.VectorSubcoreMesh(
    core_axis_name="c", subcore_axis_name="s", num_cores=NC, num_subcores=NS)

_sc_params = pltpu.CompilerParams(
    needs_layout_passes=False, use_tc_tiling_on_sc=False)


@functools.partial(
    pl.kernel,
    mesh=_mesh,
    out_type=jax.ShapeDtypeStruct((NW, N), jnp.float32),
    scratch_types=[
        pltpu.VMEM((N,), jnp.float32),
        pltpu.VMEM((NCHUNK, K), jnp.int32),
        pltpu.VMEM((NCHUNK, K), jnp.float32),
    ],
    compiler_params=_sc_params,
)
def _sc_degree(dst_hbm, ew_hbm, out_hbm, degbuf, didx, ewa):
    c = lax.axis_index("c")
    s = lax.axis_index("s")
    wid = s * NC + c
    zero = jnp.zeros((L,), jnp.float32)

    pltpu.sync_copy(dst_hbm.at[wid], didx)
    pltpu.sync_copy(ew_hbm.at[wid], ewa)

    @pl.loop(0, N // L, unroll=8)
    def _zero(i):
        degbuf[pl.ds(i * L, L)] = zero

    @pl.loop(0, NCHUNK)
    def _chunk(cnk):
        @pl.loop(0, K // L, unroll=8)
        def _grp(g):
            dv = didx[cnk, pl.ds(g * L, L)]
            ev = ewa[cnk, pl.ds(g * L, L)]
            plsc.addupdate_scatter(degbuf, [dv], ev)

    pltpu.sync_copy(degbuf, out_hbm.at[wid])


@functools.partial(
    pl.kernel,
    mesh=_mesh,
    out_type=jax.ShapeDtypeStruct((NC, N, H), jnp.float32),
    scratch_types=[
        pltpu.VMEM_SHARED((N, H), jnp.float32),
        pltpu.VMEM_SHARED((N, H // 2), jnp.float32),
        [pltpu.VMEM((K, H // 2), jnp.float32)] * 4,
        [pltpu.VMEM((K, H), jnp.float32)] * 2,
        pltpu.VMEM((NCHUNK, K), jnp.int32),
        pltpu.VMEM((NCHUNK, K), jnp.int32),
        pltpu.VMEM((NCHUNK, K), jnp.float32),
        [pltpu.SemaphoreType.DMA] * 4,
        [pltpu.SemaphoreType.DMA] * 2,
    ],
    compiler_params=_sc_params,
)
def _sc_propagate(src_hbm, dst_hbm, ew_hbm, hp_hbm, out_hbm,
                  zsh, tsh, rows_bufs, fbufs, sidx, didx, ewa, gsems, ssems):
    c = lax.axis_index("c")
    s = lax.axis_index("s")
    wid = s * NC + c
    zero = jnp.zeros((L,), jnp.float32)

    pltpu.sync_copy(src_hbm.at[wid], sidx)
    pltpu.sync_copy(dst_hbm.at[wid], didx)
    pltpu.sync_copy(ew_hbm.at[wid], ewa)

    pltpu.sync_copy(hp_hbm.at[pl.ds(s * RPT, RPT)],
                    tsh.at[pl.ds(s * RPT, RPT)])

    zb = fbufs[1]

    @pl.loop(0, K, unroll=8)
    def _zeroloop(r):
        for j in range(H // L):
            zb[r, pl.ds(j * L, L)] = zero

    for k in range(5):
        pltpu.sync_copy(zb.at[pl.ds(0, RPT // 5)],
                        zsh.at[pl.ds(s * RPT + k * (RPT // 5), RPT // 5)])

    plsc.subcore_barrier()

    for b in range(3):
        pltpu.async_copy(tsh.at[sidx.at[b]], rows_bufs[b], gsems[b])

    @pl.loop(0, NCHUNK, step=4)
    def _quad(c0):
        for b in range(4):
            cc = c0 + b
            rows = rows_bufs[b]
            fb = fbufs[b % 2]
            pltpu.make_async_copy(hp_hbm.at[sidx.at[0]], rows,
                                  gsems[b]).wait()

            @pl.when(cc >= 2)
            def _retire():
                pltpu.make_async_copy(out_hbm.at[c, pl.ds(0, K)], fb,
                                      ssems[b % 2]).wait()

            idx_c = jnp.full((L,), cc, jnp.int32)

            @pl.loop(0, K, unroll=8)
            def _scale(e):
                w = plsc.load_gather(ewa, [idx_c, jnp.full((L,), e, jnp.int32)])
                for g in range(H // 32):
                    pw = rows[e, pl.ds(g * L, L)]
                    pb = plsc.bitcast(pw, jnp.bfloat16)
                    va, vb = plsc.unpack(pb, format=plsc.PackFormat.INTERLEAVED)
                    fb[e, pl.ds(g * 32, L)] = va * w
                    fb[e, pl.ds(g * 32 + L, L)] = vb * w

            pltpu.async_copy(fb, zsh.at[didx.at[cc]], ssems[b % 2], add=True)

            bn = (b + 3) % 4
            nxt = cc + 3

            @pl.when(nxt < NCHUNK)
            def _prefetch():
                pltpu.async_copy(tsh.at[sidx.at[nxt]], rows_bufs[bn],
                                 gsems[bn])

    for p in range(2):
        pltpu.make_async_copy(out_hbm.at[c, pl.ds(0, K)], fbufs[p],
                              ssems[p]).wait()

    plsc.subcore_barrier()
    pltpu.sync_copy(zsh.at[pl.ds(s * RPT, RPT)],
                    out_hbm.at[c, pl.ds(s * RPT, RPT)])


R = 1024
GRID = (N + R - 1) // R

_ALPHA = 1.6732632423543772848170429916717
_SCALE = 1.0507009873554804934193349852946


def _selu(x):
    return _SCALE * jnp.where(x > 0, x, _ALPHA * (jnp.exp(x) - 1.0))


def _tc1_body(dp_ref, x_ref, w_ref, hp_ref, dinv_ref):
    deg = jnp.sum(dp_ref[...], axis=0) + 1.0
    dinv = jnp.where(deg > 0, lax.rsqrt(deg), 0.0)
    h = jnp.dot(x_ref[...], w_ref[...], preferred_element_type=jnp.float32)
    hp_ref[...] = h * dinv[:, None]
    dinv_ref[...] = dinv


def _tc1(deg_parts, x, gW1):
    return pl.pallas_call(
        _tc1_body,
        grid=(GRID,),
        in_specs=[
            pl.BlockSpec((NW, R), lambda i: (0, i)),
            pl.BlockSpec((R, D_IN), lambda i: (i, 0)),
            pl.BlockSpec((D_IN, H), lambda i: (0, 0)),
        ],
        out_specs=[
            pl.BlockSpec((R, H), lambda i: (i, 0)),
            pl.BlockSpec((R,), lambda i: (i,)),
        ],
        out_shape=[
            jax.ShapeDtypeStruct((N, H), jnp.float32),
            jax.ShapeDtypeStruct((N,), jnp.float32),
        ],
    )(deg_parts, x, gW1)


def _tc2_body(z_ref, hp_ref, dinv_ref, b1_ref, w2_ref, gp_ref):
    z = z_ref[0] + z_ref[1]
    dinv = dinv_ref[...]
    h2 = _selu((z + hp_ref[...]) * dinv[:, None] + b1_ref[...])
    g = jnp.dot(h2, w2_ref[...], preferred_element_type=jnp.float32)
    gp_ref[...] = g * dinv[:, None]


def _tc2(z_parts, hp, dinv, gb1, gW2):
    return pl.pallas_call(
        _tc2_body,
        grid=(GRID,),
        in_specs=[
            pl.BlockSpec((NC, R, H), lambda i: (0, i, 0)),
            pl.BlockSpec((R, H), lambda i: (i, 0)),
            pl.BlockSpec((R,), lambda i: (i,)),
            pl.BlockSpec((1, H), lambda i: (0, 0)),
            pl.BlockSpec((H, H), lambda i: (0, 0)),
        ],
        out_specs=pl.BlockSpec((R, H), lambda i: (i, 0)),
        out_shape=jax.ShapeDtypeStruct((N, H), jnp.float32),
    )(z_parts, hp, dinv, gb1.reshape(1, H), gW2)


def _tc3_body(z_ref, gp_ref, dinv_ref, b2_ref,
              w0_ref, b0_ref, w1_ref, b1_ref, w2_ref, b2m_ref, w3_ref,
              b3_ref, out_ref):
    z = z_ref[0] + z_ref[1]
    dinv = dinv_ref[...]
    h = _selu((z + gp_ref[...]) * dinv[:, None] + b2_ref[...])
    h = _selu(jnp.dot(h, w0_ref[...], preferred_element_type=jnp.float32)
              + b0_ref[...])
    h = _selu(jnp.dot(h, w1_ref[...], preferred_element_type=jnp.float32)
              + b1_ref[...])
    h = _selu(jnp.dot(h, w2_ref[...], preferred_element_type=jnp.float32)
              + b2m_ref[...])
    out_ref[...] = (jnp.dot(h, w3_ref[...], preferred_element_type=jnp.float32)
                    + b3_ref[...])


def _tc3(z_parts, gp, dinv, gb2, lW0, lb0, lW1, lb1, lW2, lb2, lW3, lb3):
    wspec = pl.BlockSpec((H, H), lambda i: (0, 0))
    bspec = pl.BlockSpec((1, H), lambda i: (0, 0))
    return pl.pallas_call(
        _tc3_body,
        grid=(GRID,),
        in_specs=[
            pl.BlockSpec((NC, R, H), lambda i: (0, i, 0)),
            pl.BlockSpec((R, H), lambda i: (i, 0)),
            pl.BlockSpec((R,), lambda i: (i,)),
            bspec, wspec, bspec, wspec, bspec, wspec, bspec,
            pl.BlockSpec((H, D_OUT), lambda i: (0, 0)),
            pl.BlockSpec((1, D_OUT), lambda i: (0, 0)),
        ],
        out_specs=pl.BlockSpec((R, D_OUT), lambda i: (i, 0)),
        out_shape=jax.ShapeDtypeStruct((N, D_OUT), jnp.float32),
    )(z_parts, gp, dinv, gb2.reshape(1, H),
      lW0, lb0.reshape(1, H), lW1, lb1.reshape(1, H), lW2, lb2.reshape(1, H),
      lW3, lb3.reshape(1, D_OUT))


_PERM = np.array([32 * g + (i // 2) + 16 * (i % 2)
                  for g in range(2) for i in range(32)], dtype=np.int32)


def _pack_bf16(a):
    p = a.astype(jnp.bfloat16)[:, _PERM].reshape(N, H // 2, 2)
    return jax.lax.bitcast_convert_type(p, jnp.float32)


@jax.jit
def kernel(x, edge_index, edge_attrs, gW1, gb1, gW2, gb2,
           lW0, lb0, lW1, lb1, lW2, lb2, lW3, lb3):
    src = edge_index[0]
    dst = edge_index[1]

    pad = EPAD - E
    srcp = jnp.concatenate(
        [src, jnp.zeros((pad,), jnp.int32)]).reshape(NW, NCHUNK, K)
    dstp = jnp.concatenate(
        [dst, jnp.zeros((pad,), jnp.int32)]).reshape(NW, NCHUNK, K)
    ewp = jnp.concatenate(
        [edge_attrs, jnp.zeros((pad,), jnp.float32)]).reshape(NW, NCHUNK, K)

    deg_parts = _sc_degree(dstp, ewp)
    hp, dinv = _tc1(deg_parts, x, gW1)
    z1 = _sc_propagate(srcp, dstp, ewp, _pack_bf16(hp))
    gp = _tc2(z1, hp, dinv, gb1, gW2)
    z2 = _sc_propagate(srcp, dstp, ewp, _pack_bf16(gp))
    out = _tc3(z2, gp, dinv, gb2, lW0, lb0, lW1, lb1, lW2, lb2, lW3, lb3)
    return out


# speedup vs baseline: 25.3412x; 1.0626x over previous
import functools
import numpy as np
import jax
import jax.numpy as jnp
from jax import lax
from jax.experimental import pallas as pl
from jax.experimental.pallas import tpu as pltpu
from jax.experimental.pallas import tpu_sc as plsc

N = 10000
E = 320000
D_IN = 128
H = 64
D_OUT = 128

NC = 2
NS = 16
NW = NC * NS
L = 16

K = 128
NCHUNK = 80
EPW = K * NCHUNK
EPAD = NW * EPW
RPT = N // NS

_mesh = plsc.VectorSubcoreMesh(
    core_axis_name="c", subcore_axis_name="s", num_cores=NC, num_subcores=NS)

_sc_params = pltpu.CompilerParams(
    needs_layout_passes=False, use_tc_tiling_on_sc=False)


@functools.partial(
    pl.kernel,
    mesh=_mesh,
    out_type=jax.ShapeDtypeStruct((NW, N), jnp.float32),
    scratch_types=[
        pltpu.VMEM((N,), jnp.float32),
        pltpu.VMEM((NCHUNK, K), jnp.int32),
        pltpu.VMEM((NCHUNK, K), jnp.float32),
    ],
    compiler_params=_sc_params,
)
def _sc_degree(dst_hbm, ew_hbm, out_hbm, degbuf, didx, ewa):
    c = lax.axis_index("c")
    s = lax.axis_index("s")
    wid = s * NC + c
    zero = jnp.zeros((L,), jnp.float32)

    pltpu.sync_copy(dst_hbm.at[wid], didx)
    pltpu.sync_copy(ew_hbm.at[wid], ewa)

    @pl.loop(0, N // L, unroll=8)
    def _zero(i):
        degbuf[pl.ds(i * L, L)] = zero

    @pl.loop(0, NCHUNK)
    def _chunk(cnk):
        @pl.loop(0, K // L, unroll=8)
        def _grp(g):
            dv = didx[cnk, pl.ds(g * L, L)]
            ev = ewa[cnk, pl.ds(g * L, L)]
            plsc.addupdate_scatter(degbuf, [dv], ev)

    pltpu.sync_copy(degbuf, out_hbm.at[wid])


@functools.partial(
    pl.kernel,
    mesh=_mesh,
    out_type=jax.ShapeDtypeStruct((NC, N, H), jnp.float32),
    scratch_types=[
        pltpu.VMEM_SHARED((N, H), jnp.float32),
        pltpu.VMEM_SHARED((N, H // 2), jnp.float32),
        [pltpu.VMEM((K, H // 2), jnp.float32)] * 4,
        [pltpu.VMEM((K, H), jnp.float32)] * 2,
        pltpu.VMEM((NCHUNK, K), jnp.int32),
        pltpu.VMEM((NCHUNK, K), jnp.int32),
        pltpu.VMEM((NCHUNK, K), jnp.float32),
        [pltpu.SemaphoreType.DMA] * 4,
        [pltpu.SemaphoreType.DMA] * 2,
    ],
    compiler_params=_sc_params,
)
def _sc_propagate(src_hbm, dst_hbm, ew_hbm, hp_hbm, out_hbm,
                  zsh, tsh, rows_bufs, fbufs, sidx, didx, ewa, gsems, ssems):
    c = lax.axis_index("c")
    s = lax.axis_index("s")
    wid = s * NC + c
    zero = jnp.zeros((L,), jnp.float32)

    pltpu.sync_copy(src_hbm.at[wid], sidx)
    pltpu.sync_copy(dst_hbm.at[wid], didx)
    pltpu.sync_copy(ew_hbm.at[wid], ewa)

    pltpu.sync_copy(hp_hbm.at[pl.ds(s * RPT, RPT)],
                    tsh.at[pl.ds(s * RPT, RPT)])

    zb = fbufs[1]

    @pl.loop(0, K, unroll=8)
    def _zeroloop(r):
        for j in range(H // L):
            zb[r, pl.ds(j * L, L)] = zero

    for k in range(5):
        pltpu.sync_copy(zb.at[pl.ds(0, RPT // 5)],
                        zsh.at[pl.ds(s * RPT + k * (RPT // 5), RPT // 5)])

    plsc.subcore_barrier()

    for b in range(3):
        pltpu.async_copy(tsh.at[sidx.at[b]], rows_bufs[b], gsems[b])

    @pl.loop(0, NCHUNK, step=4)
    def _quad(c0):
        for b in range(4):
            cc = c0 + b
            rows = rows_bufs[b]
            fb = fbufs[b % 2]
            pltpu.make_async_copy(hp_hbm.at[sidx.at[0]], rows,
                                  gsems[b]).wait()

            @pl.when(cc >= 2)
            def _retire():
                pltpu.make_async_copy(out_hbm.at[c, pl.ds(0, K)], fb,
                                      ssems[b % 2]).wait()

            idx_c = jnp.full((L,), cc, jnp.int32)

            @pl.loop(0, K, unroll=8)
            def _scale(e):
                w = plsc.load_gather(ewa, [idx_c, jnp.full((L,), e, jnp.int32)])
                for g in range(H // 32):
                    pw = rows[e, pl.ds(g * L, L)]
                    pb = plsc.bitcast(pw, jnp.bfloat16)
                    va, vb = plsc.unpack(pb, format=plsc.PackFormat.INTERLEAVED)
                    fb[e, pl.ds(g * 32, L)] = va * w
                    fb[e, pl.ds(g * 32 + L, L)] = vb * w

            pltpu.async_copy(fb, zsh.at[didx.at[cc]], ssems[b % 2], add=True)

            bn = (b + 3) % 4
            nxt = cc + 3

            @pl.when(nxt < NCHUNK)
            def _prefetch():
                pltpu.async_copy(tsh.at[sidx.at[nxt]], rows_bufs[bn],
                                 gsems[bn])

    for p in range(2):
        pltpu.make_async_copy(out_hbm.at[c, pl.ds(0, K)], fbufs[p],
                              ssems[p]).wait()

    plsc.subcore_barrier()
    pltpu.sync_copy(zsh.at[pl.ds(s * RPT, RPT)],
                    out_hbm.at[c, pl.ds(s * RPT, RPT)])


R = 1024
GRID = (N + R - 1) // R

_ALPHA = 1.6732632423543772848170429916717
_SCALE = 1.0507009873554804934193349852946


def _selu(x):
    return _SCALE * jnp.where(x > 0, x, _ALPHA * (jnp.exp(x) - 1.0))


def _pack_rows(h):
    u = jax.lax.bitcast_convert_type(h, jnp.uint32)
    r = (u + jnp.uint32(0x7FFF) + ((u >> 16) & jnp.uint32(1))) >> 16
    words = [r[:, 32 * g:32 * g + 16] | (r[:, 32 * g + 16:32 * g + 32] << 16)
             for g in range(H // 32)]
    return jax.lax.bitcast_convert_type(jnp.concatenate(words, axis=1),
                                        jnp.float32)


def _tc1_body(dp_ref, x_ref, w_ref, hp_ref, dinv_ref, hpq_ref):
    deg = jnp.sum(dp_ref[...], axis=0) + 1.0
    dinv = jnp.where(deg > 0, lax.rsqrt(deg), 0.0)
    h = jnp.dot(x_ref[...], w_ref[...], preferred_element_type=jnp.float32)
    hp = h * dinv[:, None]
    hp_ref[...] = hp
    dinv_ref[...] = dinv
    hpq_ref[...] = _pack_rows(hp)


def _tc1(deg_parts, x, gW1):
    return pl.pallas_call(
        _tc1_body,
        grid=(GRID,),
        in_specs=[
            pl.BlockSpec((NW, R), lambda i: (0, i)),
            pl.BlockSpec((R, D_IN), lambda i: (i, 0)),
            pl.BlockSpec((D_IN, H), lambda i: (0, 0)),
        ],
        out_specs=[
            pl.BlockSpec((R, H), lambda i: (i, 0)),
            pl.BlockSpec((R,), lambda i: (i,)),
            pl.BlockSpec((R, H // 2), lambda i: (i, 0)),
        ],
        out_shape=[
            jax.ShapeDtypeStruct((N, H), jnp.float32),
            jax.ShapeDtypeStruct((N,), jnp.float32),
            jax.ShapeDtypeStruct((N, H // 2), jnp.float32),
        ],
    )(deg_parts, x, gW1)


def _tc2_body(z_ref, hp_ref, dinv_ref, b1_ref, w2_ref, gp_ref, gpq_ref):
    z = z_ref[0] + z_ref[1]
    dinv = dinv_ref[...]
    h2 = _selu((z + hp_ref[...]) * dinv[:, None] + b1_ref[...])
    g = jnp.dot(h2, w2_ref[...], preferred_element_type=jnp.float32)
    gp = g * dinv[:, None]
    gp_ref[...] = gp
    gpq_ref[...] = _pack_rows(gp)


def _tc2(z_parts, hp, dinv, gb1, gW2):
    return pl.pallas_call(
        _tc2_body,
        grid=(GRID,),
        in_specs=[
            pl.BlockSpec((NC, R, H), lambda i: (0, i, 0)),
            pl.BlockSpec((R, H), lambda i: (i, 0)),
            pl.BlockSpec((R,), lambda i: (i,)),
            pl.BlockSpec((1, H), lambda i: (0, 0)),
            pl.BlockSpec((H, H), lambda i: (0, 0)),
        ],
        out_specs=[
            pl.BlockSpec((R, H), lambda i: (i, 0)),
            pl.BlockSpec((R, H // 2), lambda i: (i, 0)),
        ],
        out_shape=[
            jax.ShapeDtypeStruct((N, H), jnp.float32),
            jax.ShapeDtypeStruct((N, H // 2), jnp.float32),
        ],
    )(z_parts, hp, dinv, gb1.reshape(1, H), gW2)


def _tc3_body(z_ref, gp_ref, dinv_ref, b2_ref,
              w0_ref, b0_ref, w1_ref, b1_ref, w2_ref, b2m_ref, w3_ref,
              b3_ref, out_ref):
    z = z_ref[0] + z_ref[1]
    dinv = dinv_ref[...]
    h = _selu((z + gp_ref[...]) * dinv[:, None] + b2_ref[...])
    h = _selu(jnp.dot(h, w0_ref[...], preferred_element_type=jnp.float32)
              + b0_ref[...])
    h = _selu(jnp.dot(h, w1_ref[...], preferred_element_type=jnp.float32)
              + b1_ref[...])
    h = _selu(jnp.dot(h, w2_ref[...], preferred_element_type=jnp.float32)
              + b2m_ref[...])
    out_ref[...] = (jnp.dot(h, w3_ref[...], preferred_element_type=jnp.float32)
                    + b3_ref[...])


def _tc3(z_parts, gp, dinv, gb2, lW0, lb0, lW1, lb1, lW2, lb2, lW3, lb3):
    wspec = pl.BlockSpec((H, H), lambda i: (0, 0))
    bspec = pl.BlockSpec((1, H), lambda i: (0, 0))
    return pl.pallas_call(
        _tc3_body,
        grid=(GRID,),
        in_specs=[
            pl.BlockSpec((NC, R, H), lambda i: (0, i, 0)),
            pl.BlockSpec((R, H), lambda i: (i, 0)),
            pl.BlockSpec((R,), lambda i: (i,)),
            bspec, wspec, bspec, wspec, bspec, wspec, bspec,
            pl.BlockSpec((H, D_OUT), lambda i: (0, 0)),
            pl.BlockSpec((1, D_OUT), lambda i: (0, 0)),
        ],
        out_specs=pl.BlockSpec((R, D_OUT), lambda i: (i, 0)),
        out_shape=jax.ShapeDtypeStruct((N, D_OUT), jnp.float32),
    )(z_parts, gp, dinv, gb2.reshape(1, H),
      lW0, lb0.reshape(1, H), lW1, lb1.reshape(1, H), lW2, lb2.reshape(1, H),
      lW3, lb3.reshape(1, D_OUT))


@jax.jit
def kernel(x, edge_index, edge_attrs, gW1, gb1, gW2, gb2,
           lW0, lb0, lW1, lb1, lW2, lb2, lW3, lb3):
    src = edge_index[0]
    dst = edge_index[1]

    pad = EPAD - E
    srcp = jnp.concatenate(
        [src, jnp.zeros((pad,), jnp.int32)]).reshape(NW, NCHUNK, K)
    dstp = jnp.concatenate(
        [dst, jnp.zeros((pad,), jnp.int32)]).reshape(NW, NCHUNK, K)
    ewp = jnp.concatenate(
        [edge_attrs, jnp.zeros((pad,), jnp.float32)]).reshape(NW, NCHUNK, K)

    deg_parts = _sc_degree(dstp, ewp)
    hp, dinv, hpq = _tc1(deg_parts, x, gW1)
    z1 = _sc_propagate(srcp, dstp, ewp, hpq)
    gp, gpq = _tc2(z1, hp, dinv, gb1, gW2)
    z2 = _sc_propagate(srcp, dstp, ewp, gpq)
    out = _tc3(z2, gp, dinv, gb2, lW0, lb0, lW1, lb1, lW2, lb2, lW3, lb3)
    return out


# speedup vs baseline: 25.3474x; 1.0002x over previous
import functools
import jax
import jax.numpy as jnp
from jax import lax
from jax.experimental import pallas as pl
from jax.experimental.pallas import tpu as pltpu
from jax.experimental.pallas import tpu_sc as plsc

N = 10000
E = 320000
D_IN = 128
H = 64
D_OUT = 128

NC = 2
NS = 16
NW = NC * NS
L = 16

K = 128
NCHUNK = 80
EPW = K * NCHUNK
EPAD = NW * EPW
RPT = N // NS

_mesh = plsc.VectorSubcoreMesh(
    core_axis_name="c", subcore_axis_name="s", num_cores=NC, num_subcores=NS)

_sc_params = pltpu.CompilerParams(
    needs_layout_passes=False, use_tc_tiling_on_sc=False)


@functools.partial(
    pl.kernel,
    mesh=_mesh,
    out_type=jax.ShapeDtypeStruct((NW, N), jnp.float32),
    scratch_types=[
        pltpu.VMEM((N,), jnp.float32),
        pltpu.VMEM((NCHUNK, K), jnp.int32),
        pltpu.VMEM((NCHUNK, K), jnp.float32),
    ],
    compiler_params=_sc_params,
)
def _sc_degree(dst_hbm, ew_hbm, out_hbm, degbuf, didx, ewa):
    c = lax.axis_index("c")
    s = lax.axis_index("s")
    wid = s * NC + c
    zero = jnp.zeros((L,), jnp.float32)

    pltpu.sync_copy(dst_hbm.at[wid], didx)
    pltpu.sync_copy(ew_hbm.at[wid], ewa)

    @pl.loop(0, N // L, unroll=8)
    def _zero(i):
        degbuf[pl.ds(i * L, L)] = zero

    @pl.loop(0, NCHUNK)
    def _chunk(cnk):
        @pl.loop(0, K // L, unroll=8)
        def _grp(g):
            dv = didx[cnk, pl.ds(g * L, L)]
            ev = ewa[cnk, pl.ds(g * L, L)]
            plsc.addupdate_scatter(degbuf, [dv], ev)

    pltpu.sync_copy(degbuf, out_hbm.at[wid])


@functools.partial(
    pl.kernel,
    mesh=_mesh,
    out_type=jax.ShapeDtypeStruct((NC, N, H), jnp.float32),
    scratch_types=[
        pltpu.VMEM_SHARED((N, H), jnp.float32),
        pltpu.VMEM_SHARED((N, H // 2), jnp.float32),
        [pltpu.VMEM((K, H // 2), jnp.float32)] * 4,
        [pltpu.VMEM((K, H), jnp.float32)] * 2,
        pltpu.VMEM((NCHUNK, K), jnp.int32),
        pltpu.VMEM((NCHUNK, K), jnp.int32),
        pltpu.VMEM((NCHUNK, K), jnp.float32),
        [pltpu.SemaphoreType.DMA] * 4,
        [pltpu.SemaphoreType.DMA] * 2,
    ],
    compiler_params=_sc_params,
)
def _sc_propagate(src_hbm, dst_hbm, ew_hbm, hp_hbm, out_hbm,
                  zsh, tsh, rows_bufs, fbufs, sidx, didx, ewa, gsems, ssems):
    c = lax.axis_index("c")
    s = lax.axis_index("s")
    wid = s * NC + c
    zero = jnp.zeros((L,), jnp.float32)

    pltpu.sync_copy(src_hbm.at[wid], sidx)
    pltpu.sync_copy(dst_hbm.at[wid], didx)
    pltpu.sync_copy(ew_hbm.at[wid], ewa)

    pltpu.sync_copy(hp_hbm.at[pl.ds(s * RPT, RPT)],
                    tsh.at[pl.ds(s * RPT, RPT)])

    zb = fbufs[1]

    @pl.loop(0, K, unroll=8)
    def _zeroloop(r):
        for j in range(H // L):
            zb[r, pl.ds(j * L, L)] = zero

    for k in range(5):
        pltpu.sync_copy(zb.at[pl.ds(0, RPT // 5)],
                        zsh.at[pl.ds(s * RPT + k * (RPT // 5), RPT // 5)])

    plsc.subcore_barrier()

    for b in range(3):
        pltpu.async_copy(tsh.at[sidx.at[b]], rows_bufs[b], gsems[b])

    @pl.loop(0, NCHUNK, step=4)
    def _quad(c0):
        for b in range(4):
            cc = c0 + b
            rows = rows_bufs[b]
            fb = fbufs[b % 2]
            pltpu.make_async_copy(hp_hbm.at[sidx.at[0]], rows,
                                  gsems[b]).wait()

            @pl.when(cc >= 2)
            def _retire():
                pltpu.make_async_copy(out_hbm.at[c, pl.ds(0, K)], fb,
                                      ssems[b % 2]).wait()

            idx_c = jnp.full((L,), cc, jnp.int32)

            @pl.loop(0, K, unroll=8)
            def _scale(e):
                w = plsc.load_gather(ewa, [idx_c, jnp.full((L,), e, jnp.int32)])
                for g in range(H // 32):
                    pw = rows[e, pl.ds(g * L, L)]
                    pb = plsc.bitcast(pw, jnp.bfloat16)
                    va, vb = plsc.unpack(pb, format=plsc.PackFormat.INTERLEAVED)
                    fb[e, pl.ds(g * 32, L)] = va * w
                    fb[e, pl.ds(g * 32 + L, L)] = vb * w

            pltpu.async_copy(fb, zsh.at[didx.at[cc]], ssems[b % 2], add=True)

            bn = (b + 3) % 4
            nxt = cc + 3

            @pl.when(nxt < NCHUNK)
            def _prefetch():
                pltpu.async_copy(tsh.at[sidx.at[nxt]], rows_bufs[bn],
                                 gsems[bn])

    for p in range(2):
        pltpu.make_async_copy(out_hbm.at[c, pl.ds(0, K)], fbufs[p],
                              ssems[p]).wait()

    plsc.subcore_barrier()
    pltpu.sync_copy(zsh.at[pl.ds(s * RPT, RPT)],
                    out_hbm.at[c, pl.ds(s * RPT, RPT)])


R = 1024
GRID = (N + R - 1) // R

_ALPHA = 1.6732632423543772848170429916717
_SCALE = 1.0507009873554804934193349852946


def _selu(x):
    return _SCALE * jnp.where(x > 0, x, _ALPHA * (jnp.exp(x) - 1.0))


def _pack_rows(h):
    u = jax.lax.bitcast_convert_type(h, jnp.uint32)
    r = (u + jnp.uint32(0x7FFF) + ((u >> 16) & jnp.uint32(1))) >> 16
    words = [r[:, 32 * g:32 * g + 16] | (r[:, 32 * g + 16:32 * g + 32] << 16)
             for g in range(H // 32)]
    return jax.lax.bitcast_convert_type(jnp.concatenate(words, axis=1),
                                        jnp.float32)


def _tc1_body(dp_ref, x_ref, w_ref, hp_ref, dinv_ref, hpq_ref):
    deg = jnp.sum(dp_ref[...], axis=0) + 1.0
    dinv = jnp.where(deg > 0, lax.rsqrt(deg), 0.0)
    h = jnp.dot(x_ref[...], w_ref[...], preferred_element_type=jnp.float32)
    hp = h * dinv[:, None]
    hp_ref[...] = hp
    dinv_ref[...] = dinv
    hpq_ref[...] = _pack_rows(hp)


def _tc1(deg_parts, x, gW1):
    return pl.pallas_call(
        _tc1_body,
        grid=(GRID,),
        in_specs=[
            pl.BlockSpec((NW, R), lambda i: (0, i)),
            pl.BlockSpec((R, D_IN), lambda i: (i, 0)),
            pl.BlockSpec((D_IN, H), lambda i: (0, 0)),
        ],
        out_specs=[
            pl.BlockSpec((R, H), lambda i: (i, 0)),
            pl.BlockSpec((R,), lambda i: (i,)),
            pl.BlockSpec((R, H // 2), lambda i: (i, 0)),
        ],
        out_shape=[
            jax.ShapeDtypeStruct((N, H), jnp.float32),
            jax.ShapeDtypeStruct((N,), jnp.float32),
            jax.ShapeDtypeStruct((N, H // 2), jnp.float32),
        ],
    )(deg_parts, x, gW1)


def _tc2_body(z_ref, hp_ref, dinv_ref, b1_ref, w2_ref, gp_ref, gpq_ref):
    z = z_ref[0] + z_ref[1]
    dinv = dinv_ref[...]
    h2 = _selu((z + hp_ref[...]) * dinv[:, None] + b1_ref[...])
    g = jnp.dot(h2, w2_ref[...], preferred_element_type=jnp.float32)
    gp = g * dinv[:, None]
    gp_ref[...] = gp
    gpq_ref[...] = _pack_rows(gp)


def _tc2(z_parts, hp, dinv, gb1, gW2):
    return pl.pallas_call(
        _tc2_body,
        grid=(GRID,),
        in_specs=[
            pl.BlockSpec((NC, R, H), lambda i: (0, i, 0)),
            pl.BlockSpec((R, H), lambda i: (i, 0)),
            pl.BlockSpec((R,), lambda i: (i,)),
            pl.BlockSpec((1, H), lambda i: (0, 0)),
            pl.BlockSpec((H, H), lambda i: (0, 0)),
        ],
        out_specs=[
            pl.BlockSpec((R, H), lambda i: (i, 0)),
            pl.BlockSpec((R, H // 2), lambda i: (i, 0)),
        ],
        out_shape=[
            jax.ShapeDtypeStruct((N, H), jnp.float32),
            jax.ShapeDtypeStruct((N, H // 2), jnp.float32),
        ],
    )(z_parts, hp, dinv, gb1.reshape(1, H), gW2)


def _tc3_body(z_ref, gp_ref, dinv_ref, b2_ref,
              w0_ref, b0_ref, w1_ref, b1_ref, w2_ref, b2m_ref, w3_ref,
              b3_ref, out_ref):
    z = z_ref[0] + z_ref[1]
    dinv = dinv_ref[...]
    h = _selu((z + gp_ref[...]) * dinv[:, None] + b2_ref[...])
    h = _selu(jnp.dot(h, w0_ref[...], preferred_element_type=jnp.float32)
              + b0_ref[...])
    h = _selu(jnp.dot(h, w1_ref[...], preferred_element_type=jnp.float32)
              + b1_ref[...])
    h = _selu(jnp.dot(h, w2_ref[...], preferred_element_type=jnp.float32)
              + b2m_ref[...])
    out_ref[...] = (jnp.dot(h, w3_ref[...], preferred_element_type=jnp.float32)
                    + b3_ref[...])


def _tc3(z_parts, gp, dinv, gb2, lW0, lb0, lW1, lb1, lW2, lb2, lW3, lb3):
    wspec = pl.BlockSpec((H, H), lambda i: (0, 0))
    bspec = pl.BlockSpec((1, H), lambda i: (0, 0))
    return pl.pallas_call(
        _tc3_body,
        grid=(GRID,),
        in_specs=[
            pl.BlockSpec((NC, R, H), lambda i: (0, i, 0)),
            pl.BlockSpec((R, H), lambda i: (i, 0)),
            pl.BlockSpec((R,), lambda i: (i,)),
            bspec, wspec, bspec, wspec, bspec, wspec, bspec,
            pl.BlockSpec((H, D_OUT), lambda i: (0, 0)),
            pl.BlockSpec((1, D_OUT), lambda i: (0, 0)),
        ],
        out_specs=pl.BlockSpec((R, D_OUT), lambda i: (i, 0)),
        out_shape=jax.ShapeDtypeStruct((N, D_OUT), jnp.float32),
    )(z_parts, gp, dinv, gb2.reshape(1, H),
      lW0, lb0.reshape(1, H), lW1, lb1.reshape(1, H), lW2, lb2.reshape(1, H),
      lW3, lb3.reshape(1, D_OUT))


@jax.jit
def kernel(x, edge_index, edge_attrs, gW1, gb1, gW2, gb2,
           lW0, lb0, lW1, lb1, lW2, lb2, lW3, lb3):
    src = edge_index[0]
    dst = edge_index[1]

    pad = EPAD - E
    srcp = jnp.concatenate(
        [src, jnp.zeros((pad,), jnp.int32)]).reshape(NW, NCHUNK, K)
    dstp = jnp.concatenate(
        [dst, jnp.zeros((pad,), jnp.int32)]).reshape(NW, NCHUNK, K)
    ewp = jnp.concatenate(
        [edge_attrs, jnp.zeros((pad,), jnp.float32)]).reshape(NW, NCHUNK, K)

    deg_parts = _sc_degree(dstp, ewp)
    hp, dinv, hpq = _tc1(deg_parts, x, gW1)
    z1 = _sc_propagate(srcp, dstp, ewp, hpq)
    gp, gpq = _tc2(z1, hp, dinv, gb1, gW2)
    z2 = _sc_propagate(srcp, dstp, ewp, gpq)
    out = _tc3(z2, gp, dinv, gb2, lW0, lb0, lW1, lb1, lW2, lb2, lW3, lb3)
    return out
